# baseline (speedup 1.0000x reference)
import jax
import jax.numpy as jnp
from jax import lax
from jax.experimental import pallas as pl
from jax.experimental.pallas import tpu as pltpu, tpu_sc as plsc

N = 100000
E = 6400000
IN_DIM = 10
HID = 64
NUM_CLASSES = 4

NC = 2
NS = 16
NW = NC * NS
KD = 2048
DEG_T = (E // KD) // NW
DEG_REM = (E // KD) % NW
K = 512
NCHUNKS = E // K
FULL_FLOOR = NCHUNKS // NS
FULL_REM = NCHUNKS % NS
HALF_FLOOR = NCHUNKS // NW
HALF_REM2 = NCHUNKS % NW

NP = 100352
NPT = NP // NS

_MESH = plsc.VectorSubcoreMesh(
    core_axis_name="c", subcore_axis_name="s", num_cores=NC, num_subcores=NS)
_SC_PARAMS = pltpu.CompilerParams(use_tc_tiling_on_sc=False)


def _deg_body(ed, zeros_h, ones_h, out, colbuf, ones_v, hist, sem):
  del sem
  c = lax.axis_index("c")
  s = lax.axis_index("s")
  w = c * NS + s
  pltpu.sync_copy(ones_h, ones_v)
  pltpu.sync_copy(zeros_h.at[pl.ds(s * NPT, NPT)], hist.at[pl.ds(s * NPT, NPT)])
  plsc.subcore_barrier()

  def step(chunk):
    e0 = pl.multiple_of(chunk * KD, KD)
    pltpu.sync_copy(ed.at[1].at[pl.ds(e0, KD)], colbuf)
    pltpu.sync_copy(ones_v, hist.at[colbuf], add=True)

  def body(t, carry):
    step(w + NW * t)
    return carry

  lax.fori_loop(0, DEG_T, body, 0)

  @pl.when(w < DEG_REM)
  def _():
    step(NW * DEG_T + w)

  plsc.subcore_barrier()
  pltpu.sync_copy(hist.at[pl.ds(s * NPT, NPT)],
                  out.at[c].at[pl.ds(s * NPT, NPT)])


_deg_call = pl.kernel(
    _deg_body,
    compiler_params=_SC_PARAMS,
    out_type=jax.ShapeDtypeStruct((NC, NP), jnp.float32),
    mesh=_MESH,
    scratch_types=[
        pltpu.VMEM((KD,), jnp.int32),
        pltpu.VMEM((KD,), jnp.float32),
        pltpu.VMEM_SHARED((NP,), jnp.float32),
        pltpu.SemaphoreType.DMA,
    ],
)


def _spmm64_body(ed, y, zeros_h, out, ib0, ib1, ib2, st0, st1, st2, acc,
                 si0, si1, si2, sg0, sg1, sg2, ss0, ss1, ss2):
  ibs, sts = [ib0, ib1, ib2], [st0, st1, st2]
  sis, sgs, sss = [si0, si1, si2], [sg0, sg1, sg2], [ss0, ss1, ss2]
  c = lax.axis_index("c")
  s = lax.axis_index("s")
  T = FULL_FLOOR + (s < FULL_REM).astype(jnp.int32)
  for j in range(2):
    g = 2 * c + j

    def src_of(t):
      e0 = pl.multiple_of((s + NS * t) * K, K)
      return ed.at[:, pl.ds(e0, K)]

    pltpu.async_copy(src_of(0), ibs[0], sis[0])
    pltpu.sync_copy(zeros_h, acc.at[pl.ds(s * NPT, NPT)])
    plsc.subcore_barrier()

    def body(t, carry):
      for k in range(3):
        k1 = (k + 1) % 3

        @pl.when(t % 3 == k)
        def _():
          @pl.when(t >= 2)
          def _():
            pltpu.make_async_copy(sts[k1], acc.at[ibs[k1].at[1]],
                                  sss[k1]).wait()

          @pl.when(t + 1 < T)
          def _():
            pltpu.async_copy(src_of(t + 1), ibs[k1], sis[k1])

          @pl.when(t < T)
          def _():
            pltpu.make_async_copy(src_of(t), ibs[k], sis[k]).wait()
            pltpu.async_copy(y.at[g].at[ibs[k].at[0]], sts[k], sgs[k]).wait()
            pltpu.async_copy(sts[k], acc.at[ibs[k].at[1]], sss[k], add=True)
      return carry

    lax.fori_loop(0, T + 2, body, 0)
    plsc.subcore_barrier()
    pltpu.sync_copy(acc.at[pl.ds(s * NPT, NPT)],
                    out.at[g].at[pl.ds(s * NPT, NPT)])
    plsc.subcore_barrier()


_SPMM_SCRATCH = (
    [pltpu.VMEM((2, K), jnp.int32)] * 3
    + [pltpu.VMEM((K, 16), jnp.float32)] * 3
    + [pltpu.VMEM_SHARED((NP, 16), jnp.float32)]
    + [pltpu.SemaphoreType.DMA] * 9
)

_spmm64_call = pl.kernel(
    _spmm64_body,
    compiler_params=_SC_PARAMS,
    out_type=jax.ShapeDtypeStruct((4, NP, 16), jnp.float32),
    mesh=_MESH,
    scratch_types=_SPMM_SCRATCH,
)


def _spmm16_body(ed, y, zeros_h, out, ib0, ib1, ib2, st0, st1, st2, acc,
                 si0, si1, si2, sg0, sg1, sg2, ss0, ss1, ss2):
  ibs, sts = [ib0, ib1, ib2], [st0, st1, st2]
  sis, sgs, sss = [si0, si1, si2], [sg0, sg1, sg2], [ss0, ss1, ss2]
  c = lax.axis_index("c")
  s = lax.axis_index("s")
  w = c * NS + s
  T = HALF_FLOOR + (w < HALF_REM2).astype(jnp.int32)

  def src_of(t):
    e0 = pl.multiple_of((w + NW * t) * K, K)
    return ed.at[:, pl.ds(e0, K)]

  pltpu.async_copy(src_of(0), ibs[0], sis[0])
  pltpu.sync_copy(zeros_h, acc.at[pl.ds(s * NPT, NPT)])
  plsc.subcore_barrier()

  def body(t, carry):
    for k in range(3):
      k1 = (k + 1) % 3

      @pl.when(t % 3 == k)
      def _():
        @pl.when(t >= 2)
        def _():
          pltpu.make_async_copy(sts[k1], acc.at[ibs[k1].at[1]],
                                sss[k1]).wait()

        @pl.when(t + 1 < T)
        def _():
          pltpu.async_copy(src_of(t + 1), ibs[k1], sis[k1])

        @pl.when(t < T)
        def _():
          pltpu.make_async_copy(src_of(t), ibs[k], sis[k]).wait()
          pltpu.async_copy(y.at[ibs[k].at[0]], sts[k], sgs[k]).wait()
          pltpu.async_copy(sts[k], acc.at[ibs[k].at[1]], sss[k], add=True)
    return carry

  lax.fori_loop(0, T + 2, body, 0)
  plsc.subcore_barrier()
  pltpu.sync_copy(acc.at[pl.ds(s * NPT, NPT)],
                  out.at[c].at[pl.ds(s * NPT, NPT)])


_spmm16_call = pl.kernel(
    _spmm16_body,
    compiler_params=_SC_PARAMS,
    out_type=jax.ShapeDtypeStruct((NC, NP, 16), jnp.float32),
    mesh=_MESH,
    scratch_types=_SPMM_SCRATCH,
)


_R = 2000
_GRID = N // _R


def _tcA_kernel(pT, x, w1, dis_ref, y1_ref):
  deg = pT[:, 0:1] + pT[:, 1:2] + 1.0
  dis = lax.rsqrt(deg)
  dis_ref[...] = dis
  xw = jnp.dot(x[...], w1[...], preferred_element_type=jnp.float32)
  for g in range(4):
    y1_ref[g] = xw[:, g * 16:(g + 1) * 16] * dis


def _tcA(pT, x, w1):
  return pl.pallas_call(
      _tcA_kernel,
      grid=(_GRID,),
      in_specs=[
          pl.BlockSpec((_R, NC), lambda i: (i, 0)),
          pl.BlockSpec((_R, IN_DIM), lambda i: (i, 0)),
          pl.BlockSpec((IN_DIM, HID), lambda i: (0, 0)),
      ],
      out_specs=[
          pl.BlockSpec((_R, 1), lambda i: (i, 0)),
          pl.BlockSpec((4, _R, 16), lambda i: (0, i, 0)),
      ],
      out_shape=[
          jax.ShapeDtypeStruct((N, 1), jnp.float32),
          jax.ShapeDtypeStruct((4, NP, 16), jnp.float32),
      ],
  )(pT, x, w1)


def _tcMid_kernel(s_in, y_in, dis_in, b_in, w_in, ynext_ref):
  dis = dis_in[...]
  h = jnp.concatenate([s_in[g] + y_in[g] for g in range(4)], axis=1)
  h = jnp.maximum(h * dis + b_in[...], 0.0)
  xw = jnp.dot(h, w_in[...], preferred_element_type=jnp.float32)
  for g in range(4):
    ynext_ref[g] = xw[:, g * 16:(g + 1) * 16] * dis


def _tcMid(s_in, y_in, dis, b, w):
  return pl.pallas_call(
      _tcMid_kernel,
      grid=(_GRID,),
      in_specs=[
          pl.BlockSpec((4, _R, 16), lambda i: (0, i, 0)),
          pl.BlockSpec((4, _R, 16), lambda i: (0, i, 0)),
          pl.BlockSpec((_R, 1), lambda i: (i, 0)),
          pl.BlockSpec((1, HID), lambda i: (0, 0)),
          pl.BlockSpec((HID, HID), lambda i: (0, 0)),
      ],
      out_specs=pl.BlockSpec((4, _R, 16), lambda i: (0, i, 0)),
      out_shape=jax.ShapeDtypeStruct((4, NP, 16), jnp.float32),
  )(s_in, y_in, dis, b, w)


def _tcC_kernel(s_in, y_in, dis_in, b_in, w_in, y3_ref):
  dis = dis_in[...]
  h = jnp.concatenate([s_in[g] + y_in[g] for g in range(4)], axis=1)
  h = jnp.maximum(h * dis + b_in[...], 0.0)
  xw = jnp.dot(h, w_in[...], preferred_element_type=jnp.float32)
  y3_ref[...] = jnp.concatenate(
      [xw * dis, jnp.zeros((_R, 16 - NUM_CLASSES), jnp.float32)], axis=1)


def _tcC(s_in, y_in, dis, b, w):
  return pl.pallas_call(
      _tcC_kernel,
      grid=(_GRID,),
      in_specs=[
          pl.BlockSpec((4, _R, 16), lambda i: (0, i, 0)),
          pl.BlockSpec((4, _R, 16), lambda i: (0, i, 0)),
          pl.BlockSpec((_R, 1), lambda i: (i, 0)),
          pl.BlockSpec((1, HID), lambda i: (0, 0)),
          pl.BlockSpec((HID, NUM_CLASSES), lambda i: (0, 0)),
      ],
      out_specs=pl.BlockSpec((_R, 16), lambda i: (i, 0)),
      out_shape=jax.ShapeDtypeStruct((NP, 16), jnp.float32),
  )(s_in, y_in, dis, b, w)


def _tcD_kernel(t_in, y3_in, dis_in, b_in, out_ref):
  z = (t_in[0, :, 0:NUM_CLASSES] + t_in[1, :, 0:NUM_CLASSES]
       + y3_in[:, 0:NUM_CLASSES])
  z = z * dis_in[...] + b_in[...]
  m = jnp.max(z, axis=1, keepdims=True)
  u = z - m
  out_ref[...] = u - jnp.log(jnp.sum(jnp.exp(u), axis=1, keepdims=True))


def _tcD(t, y3, dis, b):
  return pl.pallas_call(
      _tcD_kernel,
      grid=(_GRID,),
      in_specs=[
          pl.BlockSpec((NC, _R, 16), lambda i: (0, i, 0)),
          pl.BlockSpec((_R, 16), lambda i: (i, 0)),
          pl.BlockSpec((_R, 1), lambda i: (i, 0)),
          pl.BlockSpec((1, NUM_CLASSES), lambda i: (0, 0)),
      ],
      out_specs=pl.BlockSpec((_R, NUM_CLASSES), lambda i: (i, 0)),
      out_shape=jax.ShapeDtypeStruct((N, NUM_CLASSES), jnp.float32),
  )(t, y3, dis, b)


def kernel(x, edge_index, W1, b1, W2, b2, W3, b3):
  zeros_hist = jnp.zeros((NP,), jnp.float32)
  zeros_acc = jnp.zeros((NPT, 16), jnp.float32)
  ones_chunk = jnp.ones((KD,), jnp.float32)

  p = _deg_call(edge_index, zeros_hist, ones_chunk)
  dis, y1 = _tcA(p.T[:N], x, W1)
  s1 = _spmm64_call(edge_index, y1, zeros_acc)
  y2 = _tcMid(s1, y1, dis, b1.reshape(1, HID), W2)
  s2 = _spmm64_call(edge_index, y2, zeros_acc)
  y3 = _tcC(s2, y2, dis, b2.reshape(1, HID), W3)
  t = _spmm16_call(edge_index, y3, zeros_acc)
  return _tcD(t, y3, dis, b3.reshape(1, NUM_CLASSES))


# speedup vs baseline: 1.2112x; 1.2112x over previous
import jax
import jax.numpy as jnp
from jax import lax
from jax.experimental import pallas as pl
from jax.experimental.pallas import tpu as pltpu, tpu_sc as plsc

N = 100000
E = 6400000
IN_DIM = 10
HID = 64
NUM_CLASSES = 4

NC = 2
NS = 16
NW = NC * NS
KD = 2048
DEG_T = (E // KD) // NW
DEG_REM = (E // KD) % NW
K = 512
NCHUNKS = E // K
FULL_FLOOR = NCHUNKS // NS
FULL_REM = NCHUNKS % NS
HALF_FLOOR = NCHUNKS // NW
HALF_REM2 = NCHUNKS % NW

NP = 100352
NPT = NP // NS

_MESH = plsc.VectorSubcoreMesh(
    core_axis_name="c", subcore_axis_name="s", num_cores=NC, num_subcores=NS)
_SC_PARAMS = pltpu.CompilerParams(use_tc_tiling_on_sc=False)


def _deg_body(ed, zeros_h, ones_h, out, colbuf, ones_v, hist, sem):
  del sem
  c = lax.axis_index("c")
  s = lax.axis_index("s")
  w = c * NS + s
  pltpu.sync_copy(ones_h, ones_v)
  pltpu.sync_copy(zeros_h.at[pl.ds(s * NPT, NPT)], hist.at[pl.ds(s * NPT, NPT)])
  plsc.subcore_barrier()

  def step(chunk):
    e0 = pl.multiple_of(chunk * KD, KD)
    pltpu.sync_copy(ed.at[1].at[pl.ds(e0, KD)], colbuf)
    pltpu.sync_copy(ones_v, hist.at[colbuf], add=True)

  def body(t, carry):
    step(w + NW * t)
    return carry

  lax.fori_loop(0, DEG_T, body, 0)

  @pl.when(w < DEG_REM)
  def _():
    step(NW * DEG_T + w)

  plsc.subcore_barrier()
  pltpu.sync_copy(hist.at[pl.ds(s * NPT, NPT)],
                  out.at[c].at[pl.ds(s * NPT, NPT)])


_deg_call = pl.kernel(
    _deg_body,
    compiler_params=_SC_PARAMS,
    out_type=jax.ShapeDtypeStruct((NC, NP), jnp.float32),
    mesh=_MESH,
    scratch_types=[
        pltpu.VMEM((KD,), jnp.int32),
        pltpu.VMEM((KD,), jnp.float32),
        pltpu.VMEM_SHARED((NP,), jnp.float32),
        pltpu.SemaphoreType.DMA,
    ],
)


def _spmm64_body(ed, y, zeros_h, out, ib0, ib1, ib2, st0, st1, st2, acc,
                 si0, si1, si2, sg0, sg1, sg2, ss0, ss1, ss2):
  c = lax.axis_index("c")
  s = lax.axis_index("s")
  T = FULL_FLOOR + (s < FULL_REM).astype(jnp.int32)
  for j in range(2):
    g = 2 * c + j

    def src_of(t):
      e0 = pl.multiple_of((s + NS * t) * K, K)
      return ed.at[:, pl.ds(e0, K)]

    def fire_idx(t, ib, si):
      @pl.when(t < T)
      def _():
        pltpu.async_copy(src_of(t), ib, si)

    def wait_idx(t, ib, si):
      pltpu.make_async_copy(src_of(t), ib, si).wait()

    for t0, ib, si in ((0, ib0, si0), (1, ib1, si1), (2, ib2, si2)):
      fire_idx(t0, ib, si)
    pltpu.sync_copy(zeros_h, acc.at[pl.ds(s * NPT, NPT)])
    plsc.subcore_barrier()

    def body(u, carry):
      t = 3 * u
      wait_idx(t, ib0, si0)
      hg0 = pltpu.async_copy(y.at[g].at[ib0.at[0]], st0, sg0)
      wait_idx(t + 1, ib1, si1)
      hg1 = pltpu.async_copy(y.at[g].at[ib1.at[0]], st1, sg1)
      hg0.wait()
      hs0 = pltpu.async_copy(st0, acc.at[ib0.at[1]], ss0, add=True)
      wait_idx(t + 2, ib2, si2)
      hg2 = pltpu.async_copy(y.at[g].at[ib2.at[0]], st2, sg2)
      hg1.wait()
      hs1 = pltpu.async_copy(st1, acc.at[ib1.at[1]], ss1, add=True)
      hs0.wait()
      fire_idx(t + 3, ib0, si0)
      hg2.wait()
      hs2 = pltpu.async_copy(st2, acc.at[ib2.at[1]], ss2, add=True)
      hs1.wait()
      fire_idx(t + 4, ib1, si1)
      hs2.wait()
      fire_idx(t + 5, ib2, si2)
      return carry

    lax.fori_loop(0, FULL_FLOOR // 3, body, 0)

    t = FULL_FLOOR - 1
    wait_idx(t, ib0, si0)
    pltpu.async_copy(y.at[g].at[ib0.at[0]], st0, sg0).wait()
    pltpu.async_copy(st0, acc.at[ib0.at[1]], ss0, add=True).wait()

    @pl.when(s < FULL_REM)
    def _():
      wait_idx(t + 1, ib1, si1)
      pltpu.async_copy(y.at[g].at[ib1.at[0]], st1, sg1).wait()
      pltpu.async_copy(st1, acc.at[ib1.at[1]], ss1, add=True).wait()

    plsc.subcore_barrier()
    pltpu.sync_copy(acc.at[pl.ds(s * NPT, NPT)],
                    out.at[g].at[pl.ds(s * NPT, NPT)])
    plsc.subcore_barrier()


_SPMM_SCRATCH = (
    [pltpu.VMEM((2, K), jnp.int32)] * 3
    + [pltpu.VMEM((K, 16), jnp.float32)] * 3
    + [pltpu.VMEM_SHARED((NP, 16), jnp.float32)]
    + [pltpu.SemaphoreType.DMA] * 9
)

_spmm64_call = pl.kernel(
    _spmm64_body,
    compiler_params=_SC_PARAMS,
    out_type=jax.ShapeDtypeStruct((4, NP, 16), jnp.float32),
    mesh=_MESH,
    scratch_types=_SPMM_SCRATCH,
)


def _spmm16_body(ed, y, zeros_h, out, ib0, ib1, ib2, st0, st1, st2, acc,
                 si0, si1, si2, sg0, sg1, sg2, ss0, ss1, ss2):
  c = lax.axis_index("c")
  s = lax.axis_index("s")
  w = c * NS + s
  T = HALF_FLOOR + (w < HALF_REM2).astype(jnp.int32)

  def src_of(t):
    e0 = pl.multiple_of((w + NW * t) * K, K)
    return ed.at[:, pl.ds(e0, K)]

  def fire_idx(t, ib, si):
    @pl.when(t < T)
    def _():
      pltpu.async_copy(src_of(t), ib, si)

  def wait_idx(t, ib, si):
    pltpu.make_async_copy(src_of(t), ib, si).wait()

  for t0, ib, si in ((0, ib0, si0), (1, ib1, si1), (2, ib2, si2)):
    fire_idx(t0, ib, si)
  pltpu.sync_copy(zeros_h, acc.at[pl.ds(s * NPT, NPT)])
  plsc.subcore_barrier()

  def body(u, carry):
    t = 3 * u
    wait_idx(t, ib0, si0)
    hg0 = pltpu.async_copy(y.at[ib0.at[0]], st0, sg0)
    wait_idx(t + 1, ib1, si1)
    hg1 = pltpu.async_copy(y.at[ib1.at[0]], st1, sg1)
    hg0.wait()
    hs0 = pltpu.async_copy(st0, acc.at[ib0.at[1]], ss0, add=True)
    wait_idx(t + 2, ib2, si2)
    hg2 = pltpu.async_copy(y.at[ib2.at[0]], st2, sg2)
    hg1.wait()
    hs1 = pltpu.async_copy(st1, acc.at[ib1.at[1]], ss1, add=True)
    hs0.wait()
    fire_idx(t + 3, ib0, si0)
    hg2.wait()
    hs2 = pltpu.async_copy(st2, acc.at[ib2.at[1]], ss2, add=True)
    hs1.wait()
    fire_idx(t + 4, ib1, si1)
    hs2.wait()
    fire_idx(t + 5, ib2, si2)
    return carry

  lax.fori_loop(0, HALF_FLOOR // 3, body, 0)

  @pl.when(w < HALF_REM2)
  def _():
    t = HALF_FLOOR
    wait_idx(t, ib0, si0)
    pltpu.async_copy(y.at[ib0.at[0]], st0, sg0).wait()
    pltpu.async_copy(st0, acc.at[ib0.at[1]], ss0, add=True).wait()

  plsc.subcore_barrier()
  pltpu.sync_copy(acc.at[pl.ds(s * NPT, NPT)],
                  out.at[c].at[pl.ds(s * NPT, NPT)])


_spmm16_call = pl.kernel(
    _spmm16_body,
    compiler_params=_SC_PARAMS,
    out_type=jax.ShapeDtypeStruct((NC, NP, 16), jnp.float32),
    mesh=_MESH,
    scratch_types=_SPMM_SCRATCH,
)


_R = 2000
_GRID = N // _R


def _tcA_kernel(pT, x, w1, dis_ref, y1_ref):
  deg = pT[:, 0:1] + pT[:, 1:2] + 1.0
  dis = lax.rsqrt(deg)
  dis_ref[...] = dis
  xw = jnp.dot(x[...], w1[...], preferred_element_type=jnp.float32)
  for g in range(4):
    y1_ref[g] = xw[:, g * 16:(g + 1) * 16] * dis


def _tcA(pT, x, w1):
  return pl.pallas_call(
      _tcA_kernel,
      grid=(_GRID,),
      in_specs=[
          pl.BlockSpec((_R, NC), lambda i: (i, 0)),
          pl.BlockSpec((_R, IN_DIM), lambda i: (i, 0)),
          pl.BlockSpec((IN_DIM, HID), lambda i: (0, 0)),
      ],
      out_specs=[
          pl.BlockSpec((_R, 1), lambda i: (i, 0)),
          pl.BlockSpec((4, _R, 16), lambda i: (0, i, 0)),
      ],
      out_shape=[
          jax.ShapeDtypeStruct((N, 1), jnp.float32),
          jax.ShapeDtypeStruct((4, NP, 16), jnp.float32),
      ],
  )(pT, x, w1)


def _tcMid_kernel(s_in, y_in, dis_in, b_in, w_in, ynext_ref):
  dis = dis_in[...]
  h = jnp.concatenate([s_in[g] + y_in[g] for g in range(4)], axis=1)
  h = jnp.maximum(h * dis + b_in[...], 0.0)
  xw = jnp.dot(h, w_in[...], preferred_element_type=jnp.float32)
  for g in range(4):
    ynext_ref[g] = xw[:, g * 16:(g + 1) * 16] * dis


def _tcMid(s_in, y_in, dis, b, w):
  return pl.pallas_call(
      _tcMid_kernel,
      grid=(_GRID,),
      in_specs=[
          pl.BlockSpec((4, _R, 16), lambda i: (0, i, 0)),
          pl.BlockSpec((4, _R, 16), lambda i: (0, i, 0)),
          pl.BlockSpec((_R, 1), lambda i: (i, 0)),
          pl.BlockSpec((1, HID), lambda i: (0, 0)),
          pl.BlockSpec((HID, HID), lambda i: (0, 0)),
      ],
      out_specs=pl.BlockSpec((4, _R, 16), lambda i: (0, i, 0)),
      out_shape=jax.ShapeDtypeStruct((4, NP, 16), jnp.float32),
  )(s_in, y_in, dis, b, w)


def _tcC_kernel(s_in, y_in, dis_in, b_in, w_in, y3_ref):
  dis = dis_in[...]
  h = jnp.concatenate([s_in[g] + y_in[g] for g in range(4)], axis=1)
  h = jnp.maximum(h * dis + b_in[...], 0.0)
  xw = jnp.dot(h, w_in[...], preferred_element_type=jnp.float32)
  y3_ref[...] = jnp.concatenate(
      [xw * dis, jnp.zeros((_R, 16 - NUM_CLASSES), jnp.float32)], axis=1)


def _tcC(s_in, y_in, dis, b, w):
  return pl.pallas_call(
      _tcC_kernel,
      grid=(_GRID,),
      in_specs=[
          pl.BlockSpec((4, _R, 16), lambda i: (0, i, 0)),
          pl.BlockSpec((4, _R, 16), lambda i: (0, i, 0)),
          pl.BlockSpec((_R, 1), lambda i: (i, 0)),
          pl.BlockSpec((1, HID), lambda i: (0, 0)),
          pl.BlockSpec((HID, NUM_CLASSES), lambda i: (0, 0)),
      ],
      out_specs=pl.BlockSpec((_R, 16), lambda i: (i, 0)),
      out_shape=jax.ShapeDtypeStruct((NP, 16), jnp.float32),
  )(s_in, y_in, dis, b, w)


def _tcD_kernel(t_in, y3_in, dis_in, b_in, out_ref):
  z = (t_in[0, :, 0:NUM_CLASSES] + t_in[1, :, 0:NUM_CLASSES]
       + y3_in[:, 0:NUM_CLASSES])
  z = z * dis_in[...] + b_in[...]
  m = jnp.max(z, axis=1, keepdims=True)
  u = z - m
  out_ref[...] = u - jnp.log(jnp.sum(jnp.exp(u), axis=1, keepdims=True))


def _tcD(t, y3, dis, b):
  return pl.pallas_call(
      _tcD_kernel,
      grid=(_GRID,),
      in_specs=[
          pl.BlockSpec((NC, _R, 16), lambda i: (0, i, 0)),
          pl.BlockSpec((_R, 16), lambda i: (i, 0)),
          pl.BlockSpec((_R, 1), lambda i: (i, 0)),
          pl.BlockSpec((1, NUM_CLASSES), lambda i: (0, 0)),
      ],
      out_specs=pl.BlockSpec((_R, NUM_CLASSES), lambda i: (i, 0)),
      out_shape=jax.ShapeDtypeStruct((N, NUM_CLASSES), jnp.float32),
  )(t, y3, dis, b)


def kernel(x, edge_index, W1, b1, W2, b2, W3, b3):
  zeros_hist = jnp.zeros((NP,), jnp.float32)
  zeros_acc = jnp.zeros((NPT, 16), jnp.float32)
  ones_chunk = jnp.ones((KD,), jnp.float32)

  p = _deg_call(edge_index, zeros_hist, ones_chunk)
  dis, y1 = _tcA(p.T[:N], x, W1)
  s1 = _spmm64_call(edge_index, y1, zeros_acc)
  y2 = _tcMid(s1, y1, dis, b1.reshape(1, HID), W2)
  s2 = _spmm64_call(edge_index, y2, zeros_acc)
  y3 = _tcC(s2, y2, dis, b2.reshape(1, HID), W3)
  t = _spmm16_call(edge_index, y3, zeros_acc)
  return _tcD(t, y3, dis, b3.reshape(1, NUM_CLASSES))


# speedup vs baseline: 1.7014x; 1.4047x over previous
import jax
import jax.numpy as jnp
from jax import lax
from jax.experimental import pallas as pl
from jax.experimental.pallas import tpu as pltpu, tpu_sc as plsc

N = 100000
E = 6400000
IN_DIM = 10
HID = 64
NUM_CLASSES = 4

NC = 2
NS = 16
NW = NC * NS
KD = 2048
DEG_T = (E // KD) // NW
DEG_REM = (E // KD) % NW
K = 512
NCHUNKS = E // K
FULL_FLOOR = NCHUNKS // NS
FULL_REM = NCHUNKS % NS
HALF_FLOOR = NCHUNKS // NW
HALF_REM2 = NCHUNKS % NW

NP = 100352
NPT = NP // NS

_MESH = plsc.VectorSubcoreMesh(
    core_axis_name="c", subcore_axis_name="s", num_cores=NC, num_subcores=NS)
_SC_PARAMS = pltpu.CompilerParams(use_tc_tiling_on_sc=False)


def _deg_body(ed, zeros_h, ones_h, out, colbuf, ones_v, hist, sem):
  del sem
  c = lax.axis_index("c")
  s = lax.axis_index("s")
  w = c * NS + s
  pltpu.sync_copy(ones_h, ones_v)
  pltpu.sync_copy(zeros_h.at[pl.ds(s * NPT, NPT)], hist.at[pl.ds(s * NPT, NPT)])
  plsc.subcore_barrier()

  def step(chunk):
    e0 = pl.multiple_of(chunk * KD, KD)
    pltpu.sync_copy(ed.at[1].at[pl.ds(e0, KD)], colbuf)
    pltpu.sync_copy(ones_v, hist.at[colbuf], add=True)

  def body(t, carry):
    step(w + NW * t)
    return carry

  lax.fori_loop(0, DEG_T, body, 0)

  @pl.when(w < DEG_REM)
  def _():
    step(NW * DEG_T + w)

  plsc.subcore_barrier()
  pltpu.sync_copy(hist.at[pl.ds(s * NPT, NPT)],
                  out.at[c].at[pl.ds(s * NPT, NPT)])


_deg_call = pl.kernel(
    _deg_body,
    compiler_params=_SC_PARAMS,
    out_type=jax.ShapeDtypeStruct((NC, NP), jnp.float32),
    mesh=_MESH,
    scratch_types=[
        pltpu.VMEM((KD,), jnp.int32),
        pltpu.VMEM((KD,), jnp.float32),
        pltpu.VMEM_SHARED((NP,), jnp.float32),
        pltpu.SemaphoreType.DMA,
    ],
)


def _spmm64_body(ed, y, zeros_h, out, ib0, ib1, ib2, st0, st1, st2, acc,
                 si0, si1, si2, sg0, sg1, sg2, ss0, ss1, ss2):
  c = lax.axis_index("c")
  s = lax.axis_index("s")
  T = FULL_FLOOR + (s < FULL_REM).astype(jnp.int32)
  for j in range(2):
    g = 2 * c + j

    def src_of(t):
      e0 = pl.multiple_of((s + NS * t) * K, K)
      return ed.at[:, pl.ds(e0, K)]

    def fire_idx(t, ib, si):
      @pl.when(t < T)
      def _():
        pltpu.async_copy(src_of(t), ib, si)

    def wait_idx(t, ib, si):
      pltpu.make_async_copy(src_of(t), ib, si).wait()

    for t0, ib, si in ((0, ib0, si0), (1, ib1, si1), (2, ib2, si2)):
      fire_idx(t0, ib, si)
    pltpu.sync_copy(zeros_h, acc.at[pl.ds(s * NPT, NPT)])
    plsc.subcore_barrier()

    def body(u, carry):
      t = 3 * u
      wait_idx(t, ib0, si0)
      hg0 = pltpu.async_copy(y.at[g].at[ib0.at[0]], st0, sg0)
      wait_idx(t + 1, ib1, si1)
      hg1 = pltpu.async_copy(y.at[g].at[ib1.at[0]], st1, sg1)
      hg0.wait()
      hs0 = pltpu.async_copy(st0, acc.at[ib0.at[1]], ss0, add=True)
      wait_idx(t + 2, ib2, si2)
      hg2 = pltpu.async_copy(y.at[g].at[ib2.at[0]], st2, sg2)
      hg1.wait()
      hs1 = pltpu.async_copy(st1, acc.at[ib1.at[1]], ss1, add=True)
      hs0.wait()
      fire_idx(t + 3, ib0, si0)
      hg2.wait()
      hs2 = pltpu.async_copy(st2, acc.at[ib2.at[1]], ss2, add=True)
      hs1.wait()
      fire_idx(t + 4, ib1, si1)
      hs2.wait()
      fire_idx(t + 5, ib2, si2)
      return carry

    lax.fori_loop(0, FULL_FLOOR // 3, body, 0)

    t = FULL_FLOOR - 1
    wait_idx(t, ib0, si0)
    pltpu.async_copy(y.at[g].at[ib0.at[0]], st0, sg0).wait()
    pltpu.async_copy(st0, acc.at[ib0.at[1]], ss0, add=True).wait()

    @pl.when(s < FULL_REM)
    def _():
      wait_idx(t + 1, ib1, si1)
      pltpu.async_copy(y.at[g].at[ib1.at[0]], st1, sg1).wait()
      pltpu.async_copy(st1, acc.at[ib1.at[1]], ss1, add=True).wait()

    plsc.subcore_barrier()
    pltpu.sync_copy(acc.at[pl.ds(s * NPT, NPT)],
                    out.at[g].at[pl.ds(s * NPT, NPT)])
    plsc.subcore_barrier()


_SPMM_SCRATCH = (
    [pltpu.VMEM((2, K), jnp.int32)] * 3
    + [pltpu.VMEM((K, 16), jnp.float32)] * 3
    + [pltpu.VMEM_SHARED((NP, 16), jnp.float32)]
    + [pltpu.SemaphoreType.DMA] * 9
)

_spmm64_call = pl.kernel(
    _spmm64_body,
    compiler_params=_SC_PARAMS,
    out_type=jax.ShapeDtypeStruct((4, NP, 16), jnp.float32),
    mesh=_MESH,
    scratch_types=_SPMM_SCRATCH,
)


def _spmm16_body(ed, y, zeros_h, out, ib0, ib1, ib2, st0, st1, st2, acc,
                 si0, si1, si2, sg0, sg1, sg2, ss0, ss1, ss2):
  c = lax.axis_index("c")
  s = lax.axis_index("s")
  w = c * NS + s
  T = HALF_FLOOR + (w < HALF_REM2).astype(jnp.int32)

  def src_of(t):
    e0 = pl.multiple_of((w + NW * t) * K, K)
    return ed.at[:, pl.ds(e0, K)]

  def fire_idx(t, ib, si):
    @pl.when(t < T)
    def _():
      pltpu.async_copy(src_of(t), ib, si)

  def wait_idx(t, ib, si):
    pltpu.make_async_copy(src_of(t), ib, si).wait()

  for t0, ib, si in ((0, ib0, si0), (1, ib1, si1), (2, ib2, si2)):
    fire_idx(t0, ib, si)
  pltpu.sync_copy(zeros_h, acc.at[pl.ds(s * NPT, NPT)])
  plsc.subcore_barrier()

  def body(u, carry):
    t = 3 * u
    wait_idx(t, ib0, si0)
    hg0 = pltpu.async_copy(y.at[ib0.at[0]], st0, sg0)
    wait_idx(t + 1, ib1, si1)
    hg1 = pltpu.async_copy(y.at[ib1.at[0]], st1, sg1)
    hg0.wait()
    hs0 = pltpu.async_copy(st0, acc.at[ib0.at[1]], ss0, add=True)
    wait_idx(t + 2, ib2, si2)
    hg2 = pltpu.async_copy(y.at[ib2.at[0]], st2, sg2)
    hg1.wait()
    hs1 = pltpu.async_copy(st1, acc.at[ib1.at[1]], ss1, add=True)
    hs0.wait()
    fire_idx(t + 3, ib0, si0)
    hg2.wait()
    hs2 = pltpu.async_copy(st2, acc.at[ib2.at[1]], ss2, add=True)
    hs1.wait()
    fire_idx(t + 4, ib1, si1)
    hs2.wait()
    fire_idx(t + 5, ib2, si2)
    return carry

  lax.fori_loop(0, HALF_FLOOR // 3, body, 0)

  @pl.when(w < HALF_REM2)
  def _():
    t = HALF_FLOOR
    wait_idx(t, ib0, si0)
    pltpu.async_copy(y.at[ib0.at[0]], st0, sg0).wait()
    pltpu.async_copy(st0, acc.at[ib0.at[1]], ss0, add=True).wait()

  plsc.subcore_barrier()
  pltpu.sync_copy(acc.at[pl.ds(s * NPT, NPT)],
                  out.at[c].at[pl.ds(s * NPT, NPT)])


_spmm16_call = pl.kernel(
    _spmm16_body,
    compiler_params=_SC_PARAMS,
    out_type=jax.ShapeDtypeStruct((NC, NP, 16), jnp.float32),
    mesh=_MESH,
    scratch_types=_SPMM_SCRATCH,
)


_R = 2000
_GRID = N // _R


def _tcA_kernel(pT, x, dis_ref, xs_ref):
  deg = pT[:, 0:1] + pT[:, 1:2] + 1.0
  dis = lax.rsqrt(deg)
  dis_ref[...] = dis
  xs_ref[...] = jnp.concatenate(
      [x[...] * dis, jnp.zeros((_R, 16 - IN_DIM), jnp.float32)], axis=1)


def _tcA(pT, x):
  return pl.pallas_call(
      _tcA_kernel,
      grid=(_GRID,),
      in_specs=[
          pl.BlockSpec((_R, NC), lambda i: (i, 0)),
          pl.BlockSpec((_R, IN_DIM), lambda i: (i, 0)),
      ],
      out_specs=[
          pl.BlockSpec((_R, 1), lambda i: (i, 0)),
          pl.BlockSpec((_R, 16), lambda i: (i, 0)),
      ],
      out_shape=[
          jax.ShapeDtypeStruct((N, 1), jnp.float32),
          jax.ShapeDtypeStruct((NP, 16), jnp.float32),
      ],
  )(pT, x)


def _tcB_kernel(t_in, xs_in, dis_in, w1p_in, b_in, w2_in, y2_ref):
  dis = dis_in[...]
  sx = t_in[0] + t_in[1] + xs_in[...]
  u = jnp.dot(sx, w1p_in[...], preferred_element_type=jnp.float32)
  h = jnp.maximum(u * dis + b_in[...], 0.0)
  xw = jnp.dot(h, w2_in[...], preferred_element_type=jnp.float32)
  for g in range(4):
    y2_ref[g] = xw[:, g * 16:(g + 1) * 16] * dis


def _tcB(t, xs, dis, w1p, b, w2):
  return pl.pallas_call(
      _tcB_kernel,
      grid=(_GRID,),
      in_specs=[
          pl.BlockSpec((NC, _R, 16), lambda i: (0, i, 0)),
          pl.BlockSpec((_R, 16), lambda i: (i, 0)),
          pl.BlockSpec((_R, 1), lambda i: (i, 0)),
          pl.BlockSpec((16, HID), lambda i: (0, 0)),
          pl.BlockSpec((1, HID), lambda i: (0, 0)),
          pl.BlockSpec((HID, HID), lambda i: (0, 0)),
      ],
      out_specs=pl.BlockSpec((4, _R, 16), lambda i: (0, i, 0)),
      out_shape=jax.ShapeDtypeStruct((4, NP, 16), jnp.float32),
  )(t, xs, dis, w1p, b, w2)


def _tcC_kernel(s_in, y_in, dis_in, b_in, w_in, y3_ref):
  dis = dis_in[...]
  h = jnp.concatenate([s_in[g] + y_in[g] for g in range(4)], axis=1)
  h = jnp.maximum(h * dis + b_in[...], 0.0)
  xw = jnp.dot(h, w_in[...], preferred_element_type=jnp.float32)
  y3_ref[...] = jnp.concatenate(
      [xw * dis, jnp.zeros((_R, 16 - NUM_CLASSES), jnp.float32)], axis=1)


def _tcC(s_in, y_in, dis, b, w):
  return pl.pallas_call(
      _tcC_kernel,
      grid=(_GRID,),
      in_specs=[
          pl.BlockSpec((4, _R, 16), lambda i: (0, i, 0)),
          pl.BlockSpec((4, _R, 16), lambda i: (0, i, 0)),
          pl.BlockSpec((_R, 1), lambda i: (i, 0)),
          pl.BlockSpec((1, HID), lambda i: (0, 0)),
          pl.BlockSpec((HID, NUM_CLASSES), lambda i: (0, 0)),
      ],
      out_specs=pl.BlockSpec((_R, 16), lambda i: (i, 0)),
      out_shape=jax.ShapeDtypeStruct((NP, 16), jnp.float32),
  )(s_in, y_in, dis, b, w)


def _tcD_kernel(t_in, y3_in, dis_in, b_in, out_ref):
  z = (t_in[0, :, 0:NUM_CLASSES] + t_in[1, :, 0:NUM_CLASSES]
       + y3_in[:, 0:NUM_CLASSES])
  z = z * dis_in[...] + b_in[...]
  m = jnp.max(z, axis=1, keepdims=True)
  u = z - m
  out_ref[...] = u - jnp.log(jnp.sum(jnp.exp(u), axis=1, keepdims=True))


def _tcD(t, y3, dis, b):
  return pl.pallas_call(
      _tcD_kernel,
      grid=(_GRID,),
      in_specs=[
          pl.BlockSpec((NC, _R, 16), lambda i: (0, i, 0)),
          pl.BlockSpec((_R, 16), lambda i: (i, 0)),
          pl.BlockSpec((_R, 1), lambda i: (i, 0)),
          pl.BlockSpec((1, NUM_CLASSES), lambda i: (0, 0)),
      ],
      out_specs=pl.BlockSpec((_R, NUM_CLASSES), lambda i: (i, 0)),
      out_shape=jax.ShapeDtypeStruct((N, NUM_CLASSES), jnp.float32),
  )(t, y3, dis, b)


def kernel(x, edge_index, W1, b1, W2, b2, W3, b3):
  zeros_hist = jnp.zeros((NP,), jnp.float32)
  zeros_acc = jnp.zeros((NPT, 16), jnp.float32)
  ones_chunk = jnp.ones((KD,), jnp.float32)
  w1p = jnp.concatenate([W1, jnp.zeros((16 - IN_DIM, HID), jnp.float32)])

  p = _deg_call(edge_index, zeros_hist, ones_chunk)
  dis, xs = _tcA(p.T[:N], x)
  t1 = _spmm16_call(edge_index, xs, zeros_acc)
  y2 = _tcB(t1, xs, dis, w1p, b1.reshape(1, HID), W2)
  s2 = _spmm64_call(edge_index, y2, zeros_acc)
  y3 = _tcC(s2, y2, dis, b2.reshape(1, HID), W3)
  t3 = _spmm16_call(edge_index, y3, zeros_acc)
  return _tcD(t3, y3, dis, b3.reshape(1, NUM_CLASSES))
